# baseline (speedup 1.0000x reference)
import jax
import jax.numpy as jnp
from jax import lax
from jax.experimental import pallas as pl
from jax.experimental.pallas import tpu as pltpu
from jax.experimental.pallas import tpu_sc as plsc

_D = 64
_GAMMA = 12.0
_B = 16384

_NC = 2
_NS = 16
_NW = _NC * _NS
_BPW = _B // _NW
_CHUNK = 128
_NCHUNK = _BPW // _CHUNK
_HALF = 2
_CPH = _NCHUNK // _HALF
_RPH = _BPW // _HALF
_LANES = 16


def _sc_body(h_idx, r_idx, t_idx, offs, ent, rel, out,
             hv_i, rv_i, tv_i, off_v, h_rows, r_rows, t_rows, out_v,
             sem_h, sem_r, sem_t):
  wid = lax.axis_index("s") * _NC + lax.axis_index("c")
  base = wid * _BPW

  pltpu.sync_copy(h_idx.at[pl.ds(wid * _NCHUNK, _NCHUNK), :], hv_i)
  pltpu.sync_copy(r_idx.at[pl.ds(wid * _NCHUNK, _NCHUNK), :], rv_i)
  pltpu.sync_copy(t_idx.at[pl.ds(wid * _NCHUNK, _NCHUNK), :], tv_i)
  pltpu.sync_copy(offs.at[:, pl.ds(base, _BPW)], off_v)

  lane = lax.iota(jnp.int32, _LANES)

  def half(hf):
    copies = []
    for kk in range(_CPH):
      k = hf * _CPH + kk
      dst = pl.ds(kk * _CHUNK, _CHUNK)
      copies.append(pltpu.async_copy(ent.at[hv_i.at[k]], h_rows.at[dst, :], sem_h))
      copies.append(pltpu.async_copy(rel.at[rv_i.at[k]], r_rows.at[dst, :], sem_r))
      copies.append(pltpu.async_copy(ent.at[tv_i.at[k]], t_rows.at[dst, :], sem_t))
    for c in copies:
      c.wait()

    def group(g, carry):
      rows_vec = g * _LANES + lane
      gb = pl.ds(hf * _RPH + g * _LANES, _LANES)
      oh = off_v[0, gb]
      orr = off_v[1, gb]
      ot = off_v[2, gb]
      acc = jnp.zeros((_LANES,), jnp.float32)
      for d in range(_D):
        hv = plsc.load_gather(h_rows, [rows_vec, oh + d])
        rv = plsc.load_gather(r_rows, [rows_vec, orr + d])
        tv = plsc.load_gather(t_rows, [rows_vec, ot + d])
        acc = acc + jnp.abs(hv + rv - tv)
      out_v[gb] = _GAMMA - acc
      return carry

    lax.fori_loop(0, _RPH // _LANES, group, 0)

  for hf in range(_HALF):
    half(hf)

  pltpu.sync_copy(out_v, out.at[pl.ds(base, _BPW)])


@jax.jit
def _sc_score(h_idx, r_idx, t_idx, offs, ent, rel):
  mesh = plsc.VectorSubcoreMesh(
      core_axis_name="c", subcore_axis_name="s",
      num_cores=_NC, num_subcores=_NS)
  return pl.kernel(
      _sc_body,
      out_type=jax.ShapeDtypeStruct((_B,), jnp.float32),
      mesh=mesh,
      compiler_params=pltpu.CompilerParams(
          needs_layout_passes=False, use_tc_tiling_on_sc=True),
      scratch_types=[
          pltpu.VMEM((_NCHUNK, _CHUNK), jnp.int32),
          pltpu.VMEM((_NCHUNK, _CHUNK), jnp.int32),
          pltpu.VMEM((_NCHUNK, _CHUNK), jnp.int32),
          pltpu.VMEM((3, _BPW), jnp.int32),
          pltpu.VMEM((_RPH, 2 * _D), jnp.float32),
          pltpu.VMEM((_RPH, 2 * _D), jnp.float32),
          pltpu.VMEM((_RPH, 2 * _D), jnp.float32),
          pltpu.VMEM((_BPW,), jnp.float32),
          pltpu.SemaphoreType.DMA,
          pltpu.SemaphoreType.DMA,
          pltpu.SemaphoreType.DMA,
      ],
  )(h_idx, r_idx, t_idx, offs, ent, rel)


def kernel(sample, entity_embedding, relation_embedding):
  s = sample.astype(jnp.int32)
  h, r, t = s[:, 0], s[:, 1], s[:, 2]
  ent2 = entity_embedding.reshape(entity_embedding.shape[0] // 2, 2 * _D)
  rel2 = relation_embedding.reshape(relation_embedding.shape[0] // 2, 2 * _D)
  hp = (h >> 1).reshape(_B // _CHUNK, _CHUNK)
  rp = (r >> 1).reshape(_B // _CHUNK, _CHUNK)
  tp = (t >> 1).reshape(_B // _CHUNK, _CHUNK)
  offs = jnp.stack([(h & 1) * _D, (r & 1) * _D, (t & 1) * _D], axis=0)
  score = _sc_score(hp, rp, tp, offs, ent2, rel2)
  return score.reshape(_B, 1)


# speedup vs baseline: 1.6113x; 1.6113x over previous
import jax
import jax.numpy as jnp
from jax import lax
from jax.experimental import pallas as pl
from jax.experimental.pallas import tpu as pltpu
from jax.experimental.pallas import tpu_sc as plsc

_D = 64
_GAMMA = 12.0
_B = 16384
_NE = 1000000

_NC = 2
_NS = 16
_NW = _NC * _NS
_EPW = _NE // _NW
_BW = 512
_NB = 62
_NREF = 2 * _B
_RCH = 128
_NRCH = _NREF // _RCH
_LANES = 16
_SPW = _B // _NW
_SCH = 128
_MAXOFF = 999552


def _gather_body(t3, e2, d2, lo2, hi2, base2, g_out,
                 ring, ext, evv, dvv, idxv, lov, hiv, basev,
                 sem_b, sem_s):
  wid = lax.axis_index("s") * _NC + lax.axis_index("c")
  lane = lax.iota(jnp.int32, _LANES)

  pltpu.sync_copy(lo2.at[pl.ds(wid, 1), :], lov)
  pltpu.sync_copy(hi2.at[pl.ds(wid, 1), :], hiv)
  pltpu.sync_copy(base2.at[pl.ds(wid, 1), :], basev)

  for g0 in range(_RCH // _LANES):
    idxv[0, pl.ds(g0 * _LANES, _LANES)] = jnp.full((_LANES,), -1, jnp.int32)

  base = basev[0, pl.ds(0, _LANES)][0]

  def batch_off(b):
    return jnp.minimum(base + b * _BW, _MAXOFF)

  first = pltpu.async_copy(
      t3.at[:, :, pl.ds(pl.multiple_of(batch_off(0), 128), _BW)],
      ring.at[0], sem_b)
  first.wait()

  def do_batch(b, carry):
    slot = b & 1
    off = batch_off(b)

    @pl.when(b + 1 < _NB)
    def _():
      pltpu.make_async_copy(
          t3.at[:, :, pl.ds(pl.multiple_of(batch_off(b + 1), 128), _BW)],
          ring.at[1 - slot], sem_b).start()

    lo = lov[0, pl.ds(b, _LANES)][0]
    hi = hiv[0, pl.ds(b, _LANES)][0]
    clo = lo >> 7
    chi = (hi + _RCH - 1) >> 7

    def do_chunk(c, carry2):
      pltpu.sync_copy(e2.at[pl.ds(c, 1), :], evv)
      pltpu.sync_copy(d2.at[pl.ds(c, 1), :], dvv)

      glo = jnp.maximum(lo - c * _RCH, 0) >> 4
      ghi = (jnp.minimum(hi - c * _RCH, _RCH) + _LANES - 1) >> 4

      def do_group(g, carry3):
        pos = c * _RCH + g * _LANES + lane
        m = (pos >= lo) & (pos < hi)
        e_vec = evv[0, pl.ds(g * _LANES, _LANES)]
        colb = e_vec - off
        slot_spl = jnp.full((_LANES,), slot, jnp.int32)
        row_vec = g * _LANES + lane
        for d in range(_D):
          av = jnp.full((_LANES,), d >> 3, jnp.int32)
          bv = jnp.full((_LANES,), d & 7, jnp.int32)
          val = plsc.load_gather(ring, [slot_spl, av, bv, colb], mask=m)
          plsc.store_scatter(ext, [row_vec, jnp.full((_LANES,), d, jnp.int32)],
                             val, mask=m)
        dst = jnp.where(m, dvv[0, pl.ds(g * _LANES, _LANES)], -1)
        idxv[0, pl.ds(g * _LANES, _LANES)] = dst
        return carry3

      lax.fori_loop(glo, ghi, do_group, 0)

      pltpu.async_copy(
          ext, g_out.at[plsc.Indices(idxv.at[0], ignored_value=-1)],
          sem_s).wait()
      return carry2

    lax.fori_loop(clo, chi, do_chunk, 0)

    @pl.when(b + 1 < _NB)
    def _():
      pltpu.make_async_copy(
          t3.at[:, :, pl.ds(pl.multiple_of(batch_off(b + 1), 128), _BW)],
          ring.at[1 - slot], sem_b).wait()
    return carry

  lax.fori_loop(0, _NB, do_batch, 0, unroll=False)


def _score_body(g_in, rel2, rp2, ro2, out,
                ghv, gtv, relv, rpv, rov, out_v, sem):
  wid = lax.axis_index("s") * _NC + lax.axis_index("c")
  base = wid * _SPW
  lane = lax.iota(jnp.int32, _LANES)

  pltpu.sync_copy(rel2, relv)
  pltpu.sync_copy(rp2.at[pl.ds(wid * 4, 4), :], rpv)
  pltpu.sync_copy(ro2.at[pl.ds(wid * 4, 4), :], rov)

  nch = _SPW // _SCH
  for cc in range(nch):
    s0 = base + cc * _SCH
    pltpu.sync_copy(g_in.at[pl.ds(s0, _SCH), :], ghv)
    pltpu.sync_copy(g_in.at[pl.ds(_B + s0, _SCH), :], gtv)

    def group(g, carry):
      rows_vec = g * _LANES + lane
      rp_vec = rpv[cc, pl.ds(g * _LANES, _LANES)]
      ro_vec = rov[cc, pl.ds(g * _LANES, _LANES)]
      acc = jnp.zeros((_LANES,), jnp.float32)
      for d in range(_D):
        dv = jnp.full((_LANES,), d, jnp.int32)
        hv = plsc.load_gather(ghv, [rows_vec, dv])
        tv = plsc.load_gather(gtv, [rows_vec, dv])
        rv = plsc.load_gather(relv, [rp_vec, ro_vec + d])
        acc = acc + jnp.abs(hv + rv - tv)
      out_v[pl.ds(cc * _SCH + g * _LANES, _LANES)] = _GAMMA - acc
      return carry

    lax.fori_loop(0, _SCH // _LANES, group, 0)

  pltpu.sync_copy(out_v, out.at[pl.ds(base, _SPW)])


@jax.jit
def _sc_run(t3, e2, d2, lo2, hi2, base2, rel2, rp2, ro2):
  mesh = plsc.VectorSubcoreMesh(
      core_axis_name="c", subcore_axis_name="s",
      num_cores=_NC, num_subcores=_NS)
  cp = pltpu.CompilerParams(
      needs_layout_passes=False, use_tc_tiling_on_sc=True)

  g = pl.kernel(
      _gather_body,
      out_type=jax.ShapeDtypeStruct((_NREF, 128), jnp.float32),
      mesh=mesh,
      compiler_params=cp,
      scratch_types=[
          pltpu.VMEM((2, 8, 8, _BW), jnp.float32),
          pltpu.VMEM((_RCH, 128), jnp.float32),
          pltpu.VMEM((1, _RCH), jnp.int32),
          pltpu.VMEM((1, _RCH), jnp.int32),
          pltpu.VMEM((1, _RCH), jnp.int32),
          pltpu.VMEM((1, 128), jnp.int32),
          pltpu.VMEM((1, 128), jnp.int32),
          pltpu.VMEM((1, 128), jnp.int32),
          pltpu.SemaphoreType.DMA,
          pltpu.SemaphoreType.DMA,
      ],
  )(t3, e2, d2, lo2, hi2, base2)

  return pl.kernel(
      _score_body,
      out_type=jax.ShapeDtypeStruct((_B,), jnp.float32),
      mesh=mesh,
      compiler_params=cp,
      scratch_types=[
          pltpu.VMEM((_SCH, 128), jnp.float32),
          pltpu.VMEM((_SCH, 128), jnp.float32),
          pltpu.VMEM((500, 128), jnp.float32),
          pltpu.VMEM((4, _RCH), jnp.int32),
          pltpu.VMEM((4, _RCH), jnp.int32),
          pltpu.VMEM((_SPW,), jnp.float32),
          pltpu.SemaphoreType.DMA,
      ],
  )(g, rel2, rp2, ro2)


def kernel(sample, entity_embedding, relation_embedding):
  s = sample.astype(jnp.int32)
  h, r, t = s[:, 0], s[:, 1], s[:, 2]

  t3 = entity_embedding.T.reshape(8, 8, _NE)

  e_all = jnp.concatenate([h, t])
  d_all = jnp.concatenate(
      [jnp.arange(_B, dtype=jnp.int32), jnp.arange(_B, 2 * _B, dtype=jnp.int32)])
  order = jnp.argsort(e_all)
  e_s = e_all[order]
  d_s = d_all[order]
  e2 = e_s.reshape(_NRCH, _RCH)
  d2 = d_s.reshape(_NRCH, _RCH)

  w_ids = jnp.arange(_NW, dtype=jnp.int32)
  bases = (w_ids * _EPW) & ~jnp.int32(127)
  b_ids = jnp.arange(_NB, dtype=jnp.int32)
  win_lo = jnp.minimum(bases[:, None] + b_ids[None, :] * _BW, _MAXOFF)
  low_edge = jnp.maximum(win_lo, (w_ids * _EPW)[:, None])
  flat_edges = low_edge.reshape(-1)
  lo_tab = jnp.searchsorted(e_s, flat_edges, side="left").astype(jnp.int32)
  hi_tab = jnp.concatenate(
      [lo_tab[1:], jnp.array([_NREF], jnp.int32)]).reshape(_NW, _NB)
  lo2 = lo_tab.reshape(_NW, _NB)
  pad = ((0, 0), (0, 128 - _NB))
  lo2 = jnp.pad(lo2, pad)
  hi_tab = jnp.pad(hi_tab, pad)
  base2 = jnp.pad(jnp.broadcast_to(bases[:, None], (_NW, _NB)), pad)

  rel2 = relation_embedding.reshape(relation_embedding.shape[0] // 2, 2 * _D)
  rp2 = (r >> 1).reshape(_B // _RCH, _RCH)
  ro2 = ((r & 1) * _D).reshape(_B // _RCH, _RCH)

  score = _sc_run(t3, e2, d2, lo2, hi_tab, base2, rel2, rp2, ro2)
  return score.reshape(_B, 1)
